# baseline (speedup 1.0000x reference)
import jax
import jax.numpy as jnp
from jax import lax
from jax.experimental import pallas as pl
from jax.experimental.pallas import tpu as pltpu
from jax.experimental.pallas import tpu_sc as plsc


def _sc_lookup_body(t1_hbm, emb_hbm, row_hbm, t_smem):
    cid = lax.axis_index("c")

    @pl.when(cid == 0)
    def _():
        pltpu.sync_copy(t1_hbm, t_smem)
        idx = t_smem[0]
        pltpu.sync_copy(emb_hbm.at[pl.ds(idx, 1)], row_hbm)


def _tc_add_body(x_ref, row_ref, o_ref):
    o_ref[...] = x_ref[...] + row_ref[...]


def kernel(tokens, t, emb):
    B, S, D = tokens.shape
    R = B * S
    flat = tokens.reshape(R, D)
    t1 = jnp.asarray(t, jnp.int32).reshape(1)

    mesh = plsc.ScalarSubcoreMesh(axis_name="c", num_cores=1)
    lookup = pl.kernel(
        _sc_lookup_body,
        out_type=jax.ShapeDtypeStruct((1, D), emb.dtype),
        mesh=mesh,
        scratch_types=[
            pltpu.SMEM((1,), jnp.int32),
        ],
    )
    row = lookup(t1, emb)

    BLK = 512
    out = pl.pallas_call(
        _tc_add_body,
        grid=(R // BLK,),
        in_specs=[
            pl.BlockSpec((BLK, D), lambda i: (i, 0)),
            pl.BlockSpec((1, D), lambda i: (0, 0)),
        ],
        out_specs=pl.BlockSpec((BLK, D), lambda i: (i, 0)),
        out_shape=jax.ShapeDtypeStruct((R, D), tokens.dtype),
    )(flat, row)
    return out.reshape(B, S, D)


# speedup vs baseline: 1.0164x; 1.0164x over previous
import jax
import jax.numpy as jnp
from jax import lax
from jax.experimental import pallas as pl
from jax.experimental.pallas import tpu as pltpu
from jax.experimental.pallas import tpu_sc as plsc


def _sc_lookup_body(t1_hbm, emb_hbm, row_hbm, t_smem):
    cid = lax.axis_index("c")

    @pl.when(cid == 0)
    def _():
        pltpu.sync_copy(t1_hbm, t_smem)
        idx = t_smem[0]
        pltpu.sync_copy(emb_hbm.at[pl.ds(idx, 1)], row_hbm)


def _tc_add_body(x_ref, row_ref, o_ref):
    o_ref[...] = x_ref[...] + row_ref[...]


def kernel(tokens, t, emb):
    B, S, D = tokens.shape
    R = B * S
    flat = tokens.reshape(R, D)
    t1 = jnp.asarray(t, jnp.int32).reshape(1)

    mesh = plsc.ScalarSubcoreMesh(axis_name="c", num_cores=1)
    lookup = pl.kernel(
        _sc_lookup_body,
        out_type=jax.ShapeDtypeStruct((1, D), emb.dtype),
        mesh=mesh,
        scratch_types=[
            pltpu.SMEM((1,), jnp.int32),
        ],
    )
    row = lookup(t1, emb)

    BLK = 1024
    out = pl.pallas_call(
        _tc_add_body,
        grid=(R // BLK,),
        in_specs=[
            pl.BlockSpec((BLK, D), lambda i: (i, 0)),
            pl.BlockSpec((1, D), lambda i: (0, 0)),
        ],
        out_specs=pl.BlockSpec((BLK, D), lambda i: (i, 0)),
        out_shape=jax.ShapeDtypeStruct((R, D), tokens.dtype),
    )(flat, row)
    return out.reshape(B, S, D)
